# baseline (speedup 1.0000x reference)
import functools

import jax
import jax.numpy as jnp
from jax import lax
from jax.experimental import pallas as pl
from jax.experimental.pallas import tpu as pltpu


def _round_up(n, m):
    return ((n + m - 1) // m) * m


def _lstm_kernel(x_ref, wih_ref, whh_ref, bias_ref, wfc_ref, bfc_ref,
                 out_ref, gx_sc, h_sc, c_sc, *, n_chunks, unroll):
    Tc, B_tile, F = x_ref.shape
    Hp = h_sc.shape[-1]
    t_chunk = pl.program_id(1)

    @pl.when(t_chunk == 0)
    def _():
        h_sc[...] = jnp.zeros_like(h_sc)
        c_sc[...] = jnp.zeros_like(c_sc)

    x2 = x_ref[...].reshape(Tc * B_tile, F)
    gx2 = jnp.dot(x2, wih_ref[...],
                  preferred_element_type=jnp.float32) + bias_ref[...]
    gx_sc[...] = gx2.reshape(Tc, B_tile, 4 * Hp).astype(jnp.bfloat16)

    def cell(i, _):
        gates = gx_sc[i] + jnp.dot(h_sc[...], whh_ref[...],
                                   preferred_element_type=jnp.float32)
        if_g = 0.5 * jnp.tanh(0.5 * gates[:, 0:2 * Hp]) + 0.5
        i_g = if_g[:, 0:Hp]
        f_g = if_g[:, Hp:2 * Hp]
        g_g = jnp.tanh(gates[:, 2 * Hp:3 * Hp])
        o_g = 0.5 * jnp.tanh(0.5 * gates[:, 3 * Hp:4 * Hp]) + 0.5
        c = f_g * c_sc[...] + i_g * g_g
        c_sc[...] = c
        h_sc[...] = (o_g * jnp.tanh(c)).astype(jnp.bfloat16)
        return 0

    lax.fori_loop(0, Tc, cell, 0, unroll=unroll)

    @pl.when(t_chunk == n_chunks - 1)
    def _():
        out_ref[...] = (jnp.dot(h_sc[...], wfc_ref[...],
                                preferred_element_type=jnp.float32)
                        + bfc_ref[...])


def kernel(x, w_ih, w_hh, b_ih, b_hh, w_fc, b_fc):
    x = x.astype(jnp.float32)
    B, T, F = x.shape
    H = w_hh.shape[1]
    O = w_fc.shape[0]

    Hp = _round_up(H, 128)
    Op = _round_up(O, 128)
    Bp = _round_up(B, 8)
    B_tile = min(256, Bp)
    Bp = _round_up(Bp, B_tile)

    Tc = 16
    while T % Tc:
        Tc //= 2
    n_chunks = T // Tc

    wih4 = jnp.pad(w_ih.astype(jnp.float32).reshape(4, H, F),
                   ((0, 0), (0, Hp - H), (0, 0)))
    wih_t = wih4.transpose(2, 0, 1).reshape(F, 4 * Hp).astype(jnp.bfloat16)

    whh4 = jnp.pad(w_hh.astype(jnp.float32).reshape(4, H, H),
                   ((0, 0), (0, Hp - H), (0, Hp - H)))
    whh_t = whh4.transpose(2, 0, 1).reshape(Hp, 4 * Hp).astype(jnp.bfloat16)

    bias = jnp.pad((b_ih + b_hh).astype(jnp.float32).reshape(4, H),
                   ((0, 0), (0, Hp - H))).reshape(1, 4 * Hp)

    wfc_t = jnp.pad(w_fc.astype(jnp.float32).T,
                    ((0, Hp - H), (0, Op - O))).astype(jnp.bfloat16)
    bfc = jnp.pad(b_fc.astype(jnp.float32), (0, Op - O)).reshape(1, Op)

    x_tbf = jnp.transpose(x.astype(jnp.bfloat16), (1, 0, 2))
    x_tbf = jnp.pad(x_tbf, ((0, 0), (0, Bp - B), (0, 0)))

    kern = functools.partial(_lstm_kernel, n_chunks=n_chunks, unroll=8)
    out = pl.pallas_call(
        kern,
        out_shape=jax.ShapeDtypeStruct((Bp, Op), jnp.float32),
        grid_spec=pltpu.PrefetchScalarGridSpec(
            num_scalar_prefetch=0,
            grid=(Bp // B_tile, n_chunks),
            in_specs=[
                pl.BlockSpec((Tc, B_tile, F), lambda b, t: (t, b, 0)),
                pl.BlockSpec(memory_space=pltpu.MemorySpace.VMEM),
                pl.BlockSpec(memory_space=pltpu.MemorySpace.VMEM),
                pl.BlockSpec(memory_space=pltpu.MemorySpace.VMEM),
                pl.BlockSpec(memory_space=pltpu.MemorySpace.VMEM),
                pl.BlockSpec(memory_space=pltpu.MemorySpace.VMEM),
            ],
            out_specs=pl.BlockSpec((B_tile, Op), lambda b, t: (b, 0)),
            scratch_shapes=[
                pltpu.VMEM((Tc, B_tile, 4 * Hp), jnp.bfloat16),
                pltpu.VMEM((B_tile, Hp), jnp.bfloat16),
                pltpu.VMEM((B_tile, Hp), jnp.float32),
            ],
        ),
        compiler_params=pltpu.CompilerParams(
            dimension_semantics=("parallel", "arbitrary"),
            vmem_limit_bytes=100 * 1024 * 1024,
        ),
    )(x_tbf, wih_t, whh_t, bias, wfc_t, bfc)

    return out[:B, :O]


# speedup vs baseline: 1.0449x; 1.0449x over previous
import functools

import jax
import jax.numpy as jnp
from jax import lax
from jax.experimental import pallas as pl
from jax.experimental.pallas import tpu as pltpu


def _round_up(n, m):
    return ((n + m - 1) // m) * m


def _lstm_kernel(x_ref, wih_ref, whh_ref, bias_ref, wfc_ref, bfc_ref,
                 out_ref, gx_sc, h_sc, c_sc, *, n_chunks, unroll):
    B_tile, Tc, F = x_ref.shape
    Hp = h_sc.shape[-1]
    t_chunk = pl.program_id(1)

    @pl.when(t_chunk == 0)
    def _():
        h_sc[...] = jnp.zeros_like(h_sc)
        c_sc[...] = jnp.zeros_like(c_sc)

    for j in range(Tc):
        xj = x_ref[:, j, :].astype(jnp.bfloat16)
        gx_sc[j] = jnp.dot(xj, wih_ref[...],
                           preferred_element_type=jnp.float32) + bias_ref[...]

    def cell(i, carry):
        h, c = carry
        gates = gx_sc[i] + jnp.dot(h, whh_ref[...],
                                   preferred_element_type=jnp.float32)
        if_g = 0.5 * jnp.tanh(0.5 * gates[:, 0:2 * Hp]) + 0.5
        i_g = if_g[:, 0:Hp]
        f_g = if_g[:, Hp:2 * Hp]
        g_g = jnp.tanh(gates[:, 2 * Hp:3 * Hp])
        o_g = 0.5 * jnp.tanh(0.5 * gates[:, 3 * Hp:4 * Hp]) + 0.5
        c = f_g * c + i_g * g_g
        h = (o_g * jnp.tanh(c)).astype(jnp.bfloat16)
        return h, c

    h_new, c_new = lax.fori_loop(0, Tc, cell, (h_sc[...], c_sc[...]),
                                 unroll=unroll)
    h_sc[...] = h_new
    c_sc[...] = c_new

    @pl.when(t_chunk == n_chunks - 1)
    def _():
        out_ref[...] = (jnp.dot(h_new, wfc_ref[...],
                                preferred_element_type=jnp.float32)
                        + bfc_ref[...])


def kernel(x, w_ih, w_hh, b_ih, b_hh, w_fc, b_fc):
    x = x.astype(jnp.float32)
    B, T, F = x.shape
    H = w_hh.shape[1]
    O = w_fc.shape[0]

    Hp = _round_up(H, 128)
    Op = _round_up(O, 128)
    Bp = _round_up(B, 8)
    B_tile = min(256, Bp)
    Bp = _round_up(Bp, B_tile)

    Tc = 16
    while T % Tc:
        Tc //= 2
    n_chunks = T // Tc

    wih4 = jnp.pad(w_ih.astype(jnp.float32).reshape(4, H, F),
                   ((0, 0), (0, Hp - H), (0, 0)))
    wih_t = wih4.transpose(2, 0, 1).reshape(F, 4 * Hp).astype(jnp.bfloat16)

    whh4 = jnp.pad(w_hh.astype(jnp.float32).reshape(4, H, H),
                   ((0, 0), (0, Hp - H), (0, Hp - H)))
    whh_t = whh4.transpose(2, 0, 1).reshape(Hp, 4 * Hp).astype(jnp.bfloat16)

    bias = jnp.pad((b_ih + b_hh).astype(jnp.float32).reshape(4, H),
                   ((0, 0), (0, Hp - H))).reshape(1, 4 * Hp)

    wfc_t = jnp.pad(w_fc.astype(jnp.float32).T,
                    ((0, Hp - H), (0, Op - O))).astype(jnp.bfloat16)
    bfc = jnp.pad(b_fc.astype(jnp.float32), (0, Op - O)).reshape(1, Op)

    x_btf = jnp.pad(x, ((0, Bp - B), (0, 0), (0, 0)))

    kern = functools.partial(_lstm_kernel, n_chunks=n_chunks, unroll=8)
    out = pl.pallas_call(
        kern,
        out_shape=jax.ShapeDtypeStruct((Bp, Op), jnp.float32),
        grid_spec=pltpu.PrefetchScalarGridSpec(
            num_scalar_prefetch=0,
            grid=(Bp // B_tile, n_chunks),
            in_specs=[
                pl.BlockSpec((B_tile, Tc, F), lambda b, t: (b, t, 0)),
                pl.BlockSpec(memory_space=pltpu.MemorySpace.VMEM),
                pl.BlockSpec(memory_space=pltpu.MemorySpace.VMEM),
                pl.BlockSpec(memory_space=pltpu.MemorySpace.VMEM),
                pl.BlockSpec(memory_space=pltpu.MemorySpace.VMEM),
                pl.BlockSpec(memory_space=pltpu.MemorySpace.VMEM),
            ],
            out_specs=pl.BlockSpec((B_tile, Op), lambda b, t: (b, 0)),
            scratch_shapes=[
                pltpu.VMEM((Tc, B_tile, 4 * Hp), jnp.float32),
                pltpu.VMEM((B_tile, Hp), jnp.bfloat16),
                pltpu.VMEM((B_tile, Hp), jnp.float32),
            ],
        ),
        compiler_params=pltpu.CompilerParams(
            dimension_semantics=("parallel", "arbitrary"),
            vmem_limit_bytes=100 * 1024 * 1024,
        ),
    )(x_btf, wih_t, whh_t, bias, wfc_t, bfc)

    return out[:B, :O]


# speedup vs baseline: 1.2493x; 1.1956x over previous
import functools

import jax
import jax.numpy as jnp
from jax import lax
from jax.experimental import pallas as pl
from jax.experimental.pallas import tpu as pltpu


def _round_up(n, m):
    return ((n + m - 1) // m) * m


def _lstm_kernel(x_hbm, wih_ref, whh_ref, bias_ref, wfc_ref, bfc_ref,
                 out_ref, xbuf, gx_sc, h_sc, c_sc, dma_sem,
                 *, n_chunks, unroll):
    _, Tc, B_tile, F = xbuf.shape
    Hp = h_sc.shape[-1]
    b_tile = pl.program_id(0)
    t_chunk = pl.program_id(1)
    row0 = b_tile * B_tile

    def x_copy(chunk, buf, j):
        return pltpu.make_async_copy(
            x_hbm.at[pl.ds(row0, B_tile), chunk * Tc + j, :],
            xbuf.at[buf, j], dma_sem.at[buf])

    @pl.when(t_chunk == 0)
    def _():
        h_sc[...] = jnp.zeros_like(h_sc)
        c_sc[...] = jnp.zeros_like(c_sc)
        for j in range(Tc):
            x_copy(0, 0, j).start()

    @pl.when(t_chunk + 1 < n_chunks)
    def _():
        for j in range(Tc):
            x_copy(t_chunk + 1, (t_chunk + 1) % 2, j).start()

    cur = t_chunk % 2
    for j in range(Tc):
        x_copy(t_chunk, cur, j).wait()

    x2 = xbuf[cur].reshape(Tc * B_tile, F)
    gx2 = jnp.dot(x2, wih_ref[...],
                  preferred_element_type=jnp.float32) + bias_ref[...]
    gx_sc[...] = gx2.reshape(Tc, B_tile, 4 * Hp)

    def cell(i, carry):
        h, c = carry
        gates = gx_sc[i] + jnp.dot(h, whh_ref[...],
                                   preferred_element_type=jnp.float32)
        if_g = 0.5 * jnp.tanh(0.5 * gates[:, 0:2 * Hp]) + 0.5
        i_g = if_g[:, 0:Hp]
        f_g = if_g[:, Hp:2 * Hp]
        g_g = jnp.tanh(gates[:, 2 * Hp:3 * Hp])
        o_g = 0.5 * jnp.tanh(0.5 * gates[:, 3 * Hp:4 * Hp]) + 0.5
        c = f_g * c + i_g * g_g
        h = (o_g * jnp.tanh(c)).astype(jnp.bfloat16)
        return h, c

    h_new, c_new = lax.fori_loop(0, Tc, cell, (h_sc[...], c_sc[...]),
                                 unroll=unroll)
    h_sc[...] = h_new
    c_sc[...] = c_new

    @pl.when(t_chunk == n_chunks - 1)
    def _():
        out_ref[...] = (jnp.dot(h_new, wfc_ref[...],
                                preferred_element_type=jnp.float32)
                        + bfc_ref[...])


def kernel(x, w_ih, w_hh, b_ih, b_hh, w_fc, b_fc):
    x = x.astype(jnp.float32)
    B, T, F = x.shape
    H = w_hh.shape[1]
    O = w_fc.shape[0]

    Hp = _round_up(H, 128)
    Op = _round_up(O, 128)
    Bp = _round_up(B, 8)
    B_tile = min(256, Bp)
    Bp = _round_up(Bp, B_tile)

    Tc = 16
    while T % Tc:
        Tc //= 2
    n_chunks = T // Tc

    wih4 = jnp.pad(w_ih.astype(jnp.float32).reshape(4, H, F),
                   ((0, 0), (0, Hp - H), (0, 0)))
    wih_t = wih4.transpose(2, 0, 1).reshape(F, 4 * Hp)

    whh4 = jnp.pad(w_hh.astype(jnp.float32).reshape(4, H, H),
                   ((0, 0), (0, Hp - H), (0, Hp - H)))
    whh_t = whh4.transpose(2, 0, 1).reshape(Hp, 4 * Hp).astype(jnp.bfloat16)

    bias = jnp.pad((b_ih + b_hh).astype(jnp.float32).reshape(4, H),
                   ((0, 0), (0, Hp - H))).reshape(1, 4 * Hp)

    wfc_t = jnp.pad(w_fc.astype(jnp.float32).T,
                    ((0, Hp - H), (0, Op - O))).astype(jnp.bfloat16)
    bfc = jnp.pad(b_fc.astype(jnp.float32), (0, Op - O)).reshape(1, Op)

    x_btf = jnp.pad(x, ((0, Bp - B), (0, 0), (0, 0)))

    kern = functools.partial(_lstm_kernel, n_chunks=n_chunks, unroll=8)
    out = pl.pallas_call(
        kern,
        out_shape=jax.ShapeDtypeStruct((Bp, Op), jnp.float32),
        grid_spec=pltpu.PrefetchScalarGridSpec(
            num_scalar_prefetch=0,
            grid=(Bp // B_tile, n_chunks),
            in_specs=[
                pl.BlockSpec(memory_space=pltpu.MemorySpace.HBM),
                pl.BlockSpec(memory_space=pltpu.MemorySpace.VMEM),
                pl.BlockSpec(memory_space=pltpu.MemorySpace.VMEM),
                pl.BlockSpec(memory_space=pltpu.MemorySpace.VMEM),
                pl.BlockSpec(memory_space=pltpu.MemorySpace.VMEM),
                pl.BlockSpec(memory_space=pltpu.MemorySpace.VMEM),
            ],
            out_specs=pl.BlockSpec((B_tile, Op), lambda b, t: (b, 0)),
            scratch_shapes=[
                pltpu.VMEM((2, Tc, B_tile, F), jnp.float32),
                pltpu.VMEM((Tc, B_tile, 4 * Hp), jnp.float32),
                pltpu.VMEM((B_tile, Hp), jnp.bfloat16),
                pltpu.VMEM((B_tile, Hp), jnp.float32),
                pltpu.SemaphoreType.DMA((2,)),
            ],
        ),
        compiler_params=pltpu.CompilerParams(
            dimension_semantics=("parallel", "arbitrary"),
            vmem_limit_bytes=100 * 1024 * 1024,
        ),
    )(x_btf, wih_t, whh_t, bias, wfc_t, bfc)

    return out[:B, :O]
